# baseline (speedup 1.0000x reference)
import functools

import jax
import jax.numpy as jnp
from jax import lax
from jax.experimental import pallas as pl
from jax.experimental.pallas import tpu as pltpu
from jax.experimental.pallas import tpu_sc as plsc

VOCAB = 1000
EMB = 128
B = 4096
L = 200
N = B * L

_info = plsc.get_sparse_core_info()
_NC = _info.num_cores
_NS = _info.num_subcores
NW = _NC * _NS
NB = N // NW
C = 128
NCHUNK = NB // C
NBUF = 5
PF = 2

_mesh = plsc.VectorSubcoreMesh(core_axis_name="c", subcore_axis_name="s")


@functools.partial(
    pl.kernel,
    mesh=_mesh,
    out_type=jax.ShapeDtypeStruct((N, EMB), jnp.float32),
    scratch_types=[
        pltpu.VMEM((NB,), jnp.int32),
        pltpu.VMEM((NBUF, C, EMB), jnp.float32),
        pltpu.VMEM_SHARED((VOCAB, EMB), jnp.float32),
        pltpu.SemaphoreType.DMA((NBUF,)),
        pltpu.SemaphoreType.DMA((NBUF,)),
        pltpu.SemaphoreType.DMA((NBUF,)),
    ],
)
def _visit_emb_add(idx_hbm, conc_hbm, table_hbm, out_hbm,
                   idx_v, buf_v, table_sh, ld_sem, gat_sem, st_sem):
    wid = lax.axis_index("s") * _NC + lax.axis_index("c")
    wbase = wid * NB

    @pl.when(lax.axis_index("s") == 0)
    def _stage_table():
        pltpu.sync_copy(table_hbm, table_sh)

    plsc.subcore_barrier()

    pltpu.sync_copy(idx_hbm.at[pl.ds(wbase, NB)], idx_v)

    def ld_copies(c, j):
        base = wbase + c * C
        return (
            pltpu.make_async_copy(conc_hbm.at[pl.ds(base, C)], buf_v.at[j],
                                  ld_sem.at[j]),
        )

    def st_copy(c, j):
        base = wbase + c * C
        return pltpu.make_async_copy(buf_v.at[j], out_hbm.at[pl.ds(base, C)],
                                     st_sem.at[j])

    def gat_wait(j):
        pltpu.make_async_copy(conc_hbm.at[pl.ds(0, C)], buf_v.at[j],
                              gat_sem.at[j]).wait()

    for j in range(PF):
        for cp in ld_copies(j, j):
            cp.start()

    def body(g, carry):
        for j in range(NBUF):
            c = g * NBUF + j
            jp = (j + PF) % NBUF
            jm = (j - 1) % NBUF

            @pl.when(c + PF < NCHUNK)
            def _prefetch():
                @pl.when(c >= NBUF - PF)
                def _guard():
                    st_copy(c - (NBUF - PF), jp).wait()
                for cp in ld_copies(c + PF, jp):
                    cp.start()

            for cp in ld_copies(c, j):
                cp.wait()
            pltpu.async_copy(table_sh.at[idx_v.at[pl.ds(c * C, C)]],
                             buf_v.at[j], gat_sem.at[j], add=True)

            @pl.when(c >= 1)
            def _retire_prev():
                gat_wait(jm)
                st_copy(c - 1, jm).start()
        return carry

    lax.fori_loop(0, NCHUNK // NBUF, body, 0)

    last_j = (NCHUNK - 1) % NBUF
    gat_wait(last_j)
    st_copy(NCHUNK - 1, last_j).start()
    for j in range(NBUF):
        st_copy(j, j).wait()


def kernel(visit_orders, concept_embeddings, table):
    idx = visit_orders.astype(jnp.int32).reshape(N)
    conc = concept_embeddings.reshape(N, EMB)
    out = _visit_emb_add(idx, conc, table)
    return out.reshape(B, L, EMB)


# speedup vs baseline: 1.0058x; 1.0058x over previous
import functools

import jax
import jax.numpy as jnp
from jax import lax
from jax.experimental import pallas as pl
from jax.experimental.pallas import tpu as pltpu
from jax.experimental.pallas import tpu_sc as plsc

VOCAB = 1000
EMB = 128
B = 4096
L = 200
N = B * L

_info = plsc.get_sparse_core_info()
_NC = _info.num_cores
_NS = _info.num_subcores
NW = _NC * _NS
NB = N // NW
C = 128
NCHUNK = NB // C
NBUF = 5
PF = 2

_mesh = plsc.VectorSubcoreMesh(core_axis_name="c", subcore_axis_name="s")


@functools.partial(
    pl.kernel,
    mesh=_mesh,
    out_type=jax.ShapeDtypeStruct((N, EMB), jnp.float32),
    scratch_types=[
        pltpu.VMEM((NB,), jnp.int32),
        pltpu.VMEM((NBUF, C, EMB), jnp.float32),
        pltpu.VMEM_SHARED((VOCAB, EMB), jnp.float32),
        pltpu.SemaphoreType.DMA((NBUF,)),
        pltpu.SemaphoreType.DMA((NBUF,)),
        pltpu.SemaphoreType.DMA((NBUF,)),
    ],
)
def _visit_emb_add(idx_hbm, conc_hbm, table_hbm, out_hbm,
                   idx_v, buf_v, table_sh, ld_sem, gat_sem, st_sem):
    wid = lax.axis_index("s") * _NC + lax.axis_index("c")
    wbase = wid * NB

    @pl.when(lax.axis_index("s") == 0)
    def _stage_table():
        pltpu.sync_copy(table_hbm, table_sh)

    plsc.subcore_barrier()

    pltpu.sync_copy(idx_hbm.at[pl.ds(wbase, NB)], idx_v)

    def ld_copies(c, j):
        base = wbase + c * C
        return (
            pltpu.make_async_copy(conc_hbm.at[pl.ds(base, C)], buf_v.at[j],
                                  ld_sem.at[j]),
        )

    def st_copy(c, j):
        base = wbase + c * C
        return pltpu.make_async_copy(buf_v.at[j], out_hbm.at[pl.ds(base, C)],
                                     st_sem.at[j])

    for j in range(PF):
        for cp in ld_copies(j, j):
            cp.start()

    def body(g, carry):
        for j in range(NBUF):
            c = g * NBUF + j
            jp = (j + PF) % NBUF

            @pl.when(c + PF < NCHUNK)
            def _prefetch():
                @pl.when(c >= NBUF - PF)
                def _guard():
                    st_copy(c - (NBUF - PF), jp).wait()
                for cp in ld_copies(c + PF, jp):
                    cp.start()

            for cp in ld_copies(c, j):
                cp.wait()
            pltpu.async_copy(table_sh.at[idx_v.at[pl.ds(c * C, C)]],
                             buf_v.at[j], gat_sem.at[j], add=True).wait()
            st_copy(c, j).start()
        return carry

    lax.fori_loop(0, NCHUNK // NBUF, body, 0)

    for j in range(NBUF):
        st_copy(j, j).wait()


def kernel(visit_orders, concept_embeddings, table):
    idx = visit_orders.astype(jnp.int32).reshape(N)
    conc = concept_embeddings.reshape(N, EMB)
    out = _visit_emb_add(idx, conc, table)
    return out.reshape(B, L, EMB)
